# baseline (speedup 1.0000x reference)
import functools

import jax
import jax.numpy as jnp
from jax import lax
from jax.experimental import pallas as pl
from jax.experimental.pallas import tpu as pltpu
from jax.experimental.pallas import tpu_sc as plsc

BATCH = 4096
SEQ = 200
D_EMBED = 32
NUM_WORKERS = 32
BBLK = 128
DTILES = D_EMBED // 8
K = 8
NROUNDS = SEQ // K


def _sc_gather(tokens_t, emb):
    mesh = plsc.VectorSubcoreMesh(core_axis_name="c", subcore_axis_name="s")

    @functools.partial(
        pl.kernel,
        mesh=mesh,
        out_type=jax.ShapeDtypeStruct((SEQ, DTILES, NUM_WORKERS, 1024),
                                      jnp.float32),
        scratch_types=[
            pltpu.VMEM((SEQ, BBLK), jnp.int32),
            pltpu.VMEM((K, BBLK, D_EMBED), jnp.float32),
            pltpu.VMEM((K, D_EMBED * BBLK), jnp.float32),
            pltpu.SemaphoreType.DMA((K,)),
            pltpu.SemaphoreType.DMA((K,)),
        ],
        compiler_params=pltpu.CompilerParams(
            use_tc_tiling_on_sc=False, needs_layout_passes=False),
    )
    def k(tok_hbm, emb_hbm, out_hbm, idx_v, rows_v, rt_v, gsem, ssem):
        wid = lax.axis_index("s") * 2 + lax.axis_index("c")
        pltpu.sync_copy(tok_hbm.at[:, pl.ds(wid * BBLK, BBLK)], idx_v)
        lane = jax.lax.iota(jnp.int32, 16)

        def transpose_chunk(b):
            src = rows_v.at[b]
            dst = rt_v.at[b]

            def dbody(d, carry):
                col = jnp.full((16,), 0, jnp.int32) + d
                for j0 in range(8):
                    row = lane + (j0 * 16)
                    vals = plsc.load_gather(src, [row, col])
                    dst[pl.ds(d * BBLK + j0 * 16, 16)] = vals
                return carry

            lax.fori_loop(0, D_EMBED, dbody, 0)

        def round_body(g, carry):
            base = g * K
            gathers = []
            for b in range(K):
                gathers.append(pltpu.async_copy(
                    emb_hbm.at[idx_v.at[base + b]], rows_v.at[b], gsem.at[b]))
            stores = []
            for b in range(K):
                s = base + b
                gathers[b].wait()
                transpose_chunk(b)
                for dt in range(DTILES):
                    stores.append(pltpu.async_copy(
                        rt_v.at[b].at[pl.ds(dt * 1024, 1024)],
                        out_hbm.at[s, dt, wid],
                        ssem.at[b]))
            for h in stores:
                h.wait()
            return carry

        lax.fori_loop(0, NROUNDS, round_body, 0)

    return k(tokens_t, emb)


def kernel(tokens, emb):
    out5 = _sc_gather(tokens.T, emb)
    out5 = out5.reshape(SEQ, DTILES, NUM_WORKERS, 8, BBLK)
    return out5.transpose(2, 4, 0, 1, 3).reshape(BATCH, SEQ, D_EMBED)


# speedup vs baseline: 1.4096x; 1.4096x over previous
import functools

import jax
import jax.numpy as jnp
from jax import lax
from jax.experimental import pallas as pl
from jax.experimental.pallas import tpu as pltpu
from jax.experimental.pallas import tpu_sc as plsc

BATCH = 4096
SEQ = 200
D_EMBED = 32
NUM_WORKERS = 32
BBLK = 128
DTILES = D_EMBED // 8
K = 8
NROUNDS = SEQ // K


def _sc_gather(tokens_t, emb):
    mesh = plsc.VectorSubcoreMesh(core_axis_name="c", subcore_axis_name="s")

    @functools.partial(
        pl.kernel,
        mesh=mesh,
        out_type=jax.ShapeDtypeStruct((SEQ, DTILES, NUM_WORKERS, 1024),
                                      jnp.float32),
        scratch_types=[
            pltpu.VMEM((SEQ, BBLK), jnp.int32),
            pltpu.VMEM((K, BBLK, D_EMBED), jnp.float32),
            pltpu.VMEM((K, D_EMBED * BBLK), jnp.float32),
            pltpu.SemaphoreType.DMA((K,)),
            pltpu.SemaphoreType.DMA((K,)),
        ],
        compiler_params=pltpu.CompilerParams(
            use_tc_tiling_on_sc=False, needs_layout_passes=False),
    )
    def k(tok_hbm, emb_hbm, out_hbm, idx_v, rows_v, rt_v, gsem, ssem):
        wid = lax.axis_index("s") * 2 + lax.axis_index("c")
        pltpu.sync_copy(tok_hbm.at[:, pl.ds(wid * BBLK, BBLK)], idx_v)
        lane = jax.lax.iota(jnp.int32, 16)

        base0 = lane * BBLK
        base1 = (lane + 16) * BBLK

        def transpose_chunk(b):
            src = rows_v.at[b]
            dst = rt_v.at[b]

            @plsc.parallel_loop(0, BBLK, 1, unroll=8)
            def _body(j):
                v0 = src[j, pl.ds(0, 16)]
                v1 = src[j, pl.ds(16, 16)]
                plsc.store_scatter(dst, [base0 + j], v0)
                plsc.store_scatter(dst, [base1 + j], v1)

        def round_body(g, carry):
            base = g * K
            gathers = []
            for b in range(K):
                gathers.append(pltpu.async_copy(
                    emb_hbm.at[idx_v.at[base + b]], rows_v.at[b], gsem.at[b]))
            stores = []
            for b in range(K):
                s = base + b
                gathers[b].wait()
                transpose_chunk(b)
                for dt in range(DTILES):
                    stores.append(pltpu.async_copy(
                        rt_v.at[b].at[pl.ds(dt * 1024, 1024)],
                        out_hbm.at[s, dt, wid],
                        ssem.at[b]))
            for h in stores:
                h.wait()
            return carry

        lax.fori_loop(0, NROUNDS, round_body, 0)

    return k(tokens_t, emb)


def kernel(tokens, emb):
    out5 = _sc_gather(tokens.T, emb)
    out5 = out5.reshape(SEQ, DTILES, NUM_WORKERS, 8, BBLK)
    return out5.transpose(2, 4, 0, 1, 3).reshape(BATCH, SEQ, D_EMBED)


# speedup vs baseline: 4.6093x; 3.2700x over previous
import functools

import jax
import jax.numpy as jnp
from jax import lax
from jax.experimental import pallas as pl
from jax.experimental.pallas import tpu as pltpu
from jax.experimental.pallas import tpu_sc as plsc

BATCH = 4096
SEQ = 200
D_EMBED = 32
NUM_WORKERS = 32
BBLK = 128
DTILES = D_EMBED // 8
PADW = 133
K = 8
NROUNDS = SEQ // K


def _sc_gather(tokens_t, emb):
    mesh = plsc.VectorSubcoreMesh(core_axis_name="c", subcore_axis_name="s")

    @functools.partial(
        pl.kernel,
        mesh=mesh,
        out_type=jax.ShapeDtypeStruct((SEQ, DTILES, NUM_WORKERS, 8, BBLK),
                                      jnp.float32),
        scratch_types=[
            pltpu.VMEM((SEQ, BBLK), jnp.int32),
            pltpu.VMEM((K, BBLK, D_EMBED), jnp.float32),
            pltpu.VMEM((K, D_EMBED, PADW), jnp.float32),
            pltpu.SemaphoreType.DMA((K,)),
            pltpu.SemaphoreType.DMA((K,)),
        ],
        compiler_params=pltpu.CompilerParams(
            use_tc_tiling_on_sc=False, needs_layout_passes=False),
    )
    def k(tok_hbm, emb_hbm, out_hbm, idx_v, rows_v, rt_v, gsem, ssem):
        wid = lax.axis_index("s") * 2 + lax.axis_index("c")
        pltpu.sync_copy(tok_hbm.at[:, pl.ds(wid * BBLK, BBLK)], idx_v)
        lane = jax.lax.iota(jnp.int32, 16)

        def transpose_chunk(b):
            src = rows_v.at[b]
            dst = rt_v.at[b]

            @plsc.parallel_loop(0, BBLK, 1, unroll=8)
            def _body(j):
                jvec = jnp.zeros((16,), jnp.int32) + j
                v0 = src[j, pl.ds(0, 16)]
                v1 = src[j, pl.ds(16, 16)]
                plsc.store_scatter(dst, [lane, jvec], v0)
                plsc.store_scatter(dst, [lane + 16, jvec], v1)

        def round_body(g, carry):
            base = g * K
            gathers = []
            for b in range(K):
                gathers.append(pltpu.async_copy(
                    emb_hbm.at[idx_v.at[base + b]], rows_v.at[b], gsem.at[b]))
            stores = []
            for b in range(K):
                s = base + b
                gathers[b].wait()
                transpose_chunk(b)
                for dt in range(DTILES):
                    stores.append(pltpu.async_copy(
                        rt_v.at[b].at[pl.ds(dt * 8, 8), pl.ds(0, BBLK)],
                        out_hbm.at[s, dt, wid],
                        ssem.at[b]))
            for h in stores:
                h.wait()
            return carry

        lax.fori_loop(0, NROUNDS, round_body, 0)

    return k(tokens_t, emb)


def kernel(tokens, emb):
    out5 = _sc_gather(tokens.T, emb)
    return out5.transpose(2, 4, 0, 1, 3).reshape(BATCH, SEQ, D_EMBED)


# speedup vs baseline: 4.8507x; 1.0524x over previous
import functools

import jax
import jax.numpy as jnp
from jax import lax
from jax.experimental import pallas as pl
from jax.experimental.pallas import tpu as pltpu
from jax.experimental.pallas import tpu_sc as plsc

BATCH = 4096
SEQ = 200
D_EMBED = 32
NUM_WORKERS = 32
BBLK = 128
DTILES = D_EMBED // 8
PADW = 133
K = 10
NROUNDS = SEQ // K


def _sc_gather(tokens_t, emb):
    mesh = plsc.VectorSubcoreMesh(core_axis_name="c", subcore_axis_name="s")

    @functools.partial(
        pl.kernel,
        mesh=mesh,
        out_type=jax.ShapeDtypeStruct((SEQ, DTILES, NUM_WORKERS, 8, BBLK),
                                      jnp.float32),
        scratch_types=[
            pltpu.VMEM((SEQ, BBLK), jnp.int32),
            pltpu.VMEM((K, BBLK, D_EMBED), jnp.float32),
            pltpu.VMEM((K, DTILES, 8, PADW), jnp.float32),
            pltpu.SemaphoreType.DMA((K,)),
            pltpu.SemaphoreType.DMA((K,)),
        ],
        compiler_params=pltpu.CompilerParams(
            use_tc_tiling_on_sc=False, needs_layout_passes=False),
    )
    def k(tok_hbm, emb_hbm, out_hbm, idx_v, rows_v, rt_v, gsem, ssem):
        wid = lax.axis_index("s") * 2 + lax.axis_index("c")
        pltpu.sync_copy(tok_hbm.at[:, pl.ds(wid * BBLK, BBLK)], idx_v)
        lane = jax.lax.iota(jnp.int32, 16)

        dt_lo, ds_lo = lane >> 3, lane & 7

        def transpose_chunk(b):
            src = rows_v.at[b]
            dst = rt_v.at[b]

            @plsc.parallel_loop(0, BBLK, 1, unroll=8)
            def _body(j):
                jvec = jnp.zeros((16,), jnp.int32) + j
                v0 = src[j, pl.ds(0, 16)]
                v1 = src[j, pl.ds(16, 16)]
                plsc.store_scatter(dst, [dt_lo, ds_lo, jvec], v0)
                plsc.store_scatter(dst, [dt_lo + 2, ds_lo, jvec], v1)

        def round_body(g, carry):
            base = g * K
            gathers = []
            for b in range(K):
                gathers.append(pltpu.async_copy(
                    emb_hbm.at[idx_v.at[base + b]], rows_v.at[b], gsem.at[b]))
            stores = []
            for b in range(K):
                s = base + b
                gathers[b].wait()
                transpose_chunk(b)
                stores.append(pltpu.async_copy(
                    rt_v.at[b].at[:, :, pl.ds(0, BBLK)],
                    out_hbm.at[s, :, wid],
                    ssem.at[b]))
            for h in stores:
                h.wait()
            return carry

        lax.fori_loop(0, NROUNDS, round_body, 0)

    return k(tokens_t, emb)


def kernel(tokens, emb):
    out5 = _sc_gather(tokens.T, emb)
    return out5.transpose(2, 4, 0, 1, 3).reshape(BATCH, SEQ, D_EMBED)


# speedup vs baseline: 4.8834x; 1.0067x over previous
import functools

import jax
import jax.numpy as jnp
from jax import lax
from jax.experimental import pallas as pl
from jax.experimental.pallas import tpu as pltpu
from jax.experimental.pallas import tpu_sc as plsc

BATCH = 4096
SEQ = 200
D_EMBED = 32
NUM_WORKERS = 32
BBLK = 128
DTILES = D_EMBED // 8
PADW = 133
K = 10
NROUNDS = SEQ // K


def _sc_gather(tokens_t, emb):
    mesh = plsc.VectorSubcoreMesh(core_axis_name="c", subcore_axis_name="s")

    @functools.partial(
        pl.kernel,
        mesh=mesh,
        out_type=jax.ShapeDtypeStruct((SEQ, DTILES, NUM_WORKERS, 8, BBLK),
                                      jnp.float32),
        scratch_types=[
            pltpu.VMEM((SEQ // 8, 8, BBLK), jnp.int32),
            pltpu.VMEM((K, BBLK, D_EMBED), jnp.float32),
            pltpu.VMEM((K, DTILES, 8, PADW), jnp.float32),
            pltpu.SemaphoreType.DMA((K,)),
            pltpu.SemaphoreType.DMA((K,)),
        ],
        compiler_params=pltpu.CompilerParams(
            use_tc_tiling_on_sc=False, needs_layout_passes=False),
    )
    def k(tok_hbm, emb_hbm, out_hbm, idx_v, rows_v, rt_v, gsem, ssem):
        wid = lax.axis_index("s") * 2 + lax.axis_index("c")
        pltpu.sync_copy(tok_hbm.at[:, wid], idx_v)
        lane = jax.lax.iota(jnp.int32, 16)

        dt_lo, ds_lo = lane >> 3, lane & 7

        def transpose_chunk(b):
            src = rows_v.at[b]
            dst = rt_v.at[b]

            @plsc.parallel_loop(0, BBLK, 1, unroll=8)
            def _body(j):
                jvec = jnp.zeros((16,), jnp.int32) + j
                v0 = src[j, pl.ds(0, 16)]
                v1 = src[j, pl.ds(16, 16)]
                plsc.store_scatter(dst, [dt_lo, ds_lo, jvec], v0)
                plsc.store_scatter(dst, [dt_lo + 2, ds_lo, jvec], v1)

        def round_body(g, carry):
            base = g * K
            gathers = []
            for b in range(K):
                s_ = base + b
                gathers.append(pltpu.async_copy(
                    emb_hbm.at[idx_v.at[lax.div(s_, 8), lax.rem(s_, 8)]],
                    rows_v.at[b], gsem.at[b]))
            stores = []
            for b in range(K):
                s = base + b
                gathers[b].wait()
                transpose_chunk(b)
                stores.append(pltpu.async_copy(
                    rt_v.at[b].at[:, :, pl.ds(0, BBLK)],
                    out_hbm.at[s, :, wid],
                    ssem.at[b]))
            for h in stores:
                h.wait()
            return carry

        lax.fori_loop(0, NROUNDS, round_body, 0)

    return k(tokens_t, emb)


def kernel(tokens, emb):
    tok4 = tokens.T.reshape(SEQ // 8, 8, NUM_WORKERS, BBLK).transpose(0, 2, 1, 3)
    out5 = _sc_gather(tok4, emb)
    return out5.transpose(2, 4, 0, 1, 3).reshape(BATCH, SEQ, D_EMBED)
